# baseline (speedup 1.0000x reference)
import functools

import jax
import jax.numpy as jnp
from jax import lax
from jax.experimental import pallas as pl
from jax.experimental.pallas import tpu as pltpu
from jax.experimental.pallas import tpu_sc as plsc

N = 10000
E = 320000
D = 128
NPAD = 10240
NC, NS = 2, 16
NW = NC * NS
EPW = E // NW
CH = 80
NCHUNK = EPW // CH
RPT = NPAD // NS
HRPT = 80

_mesh = plsc.VectorSubcoreMesh(core_axis_name="c", subcore_axis_name="s")
_sc_params = pltpu.CompilerParams(needs_layout_passes=False)
_sc_params_notile = pltpu.CompilerParams(needs_layout_passes=False,
                                         use_tc_tiling_on_sc=False)


@functools.partial(
    pl.kernel,
    out_type=jax.ShapeDtypeStruct((NC, NPAD), jnp.float32),
    mesh=_mesh,
    scratch_types=[
        pltpu.VMEM((HRPT, 128), jnp.int32),
        pltpu.VMEM((128,), jnp.float32),
        pltpu.VMEM((RPT,), jnp.float32),
        pltpu.VMEM_SHARED((NPAD,), jnp.float32),
    ],
    compiler_params=_sc_params,
)
def _sc_hist(col_hbm, out_hbm, colblk, ones_v, zb_v, hist_sh):
    c = lax.axis_index("c")
    s = lax.axis_index("s")
    wid = s * NC + c
    for g in range(8):
        ones_v[pl.ds(g * 16, 16)] = jnp.full((16,), 1.0, jnp.float32)
    for g in range(RPT // 16):
        zb_v[pl.ds(g * 16, 16)] = jnp.zeros((16,), jnp.float32)
    pltpu.sync_copy(col_hbm.at[pl.ds(wid * HRPT, HRPT)], colblk)
    pltpu.sync_copy(zb_v, hist_sh.at[pl.ds(s * RPT, RPT)])
    plsc.subcore_barrier()

    def body(j, carry):
        pltpu.sync_copy(ones_v, hist_sh.at[colblk.at[j]], add=True)
        return carry

    lax.fori_loop(0, HRPT, body, 0)
    plsc.subcore_barrier()
    pltpu.sync_copy(hist_sh.at[pl.ds(s * RPT, RPT)],
                    out_hbm.at[c, pl.ds(s * RPT, RPT)])


def _tc_node_body(deg2_ref, x_ref, gw_ref, gb_ref, dis_ref, sib_ref, sj_ref):
    deg = deg2_ref[0, :] + deg2_ref[1, :]
    dis_ref[...] = lax.rsqrt(jnp.maximum(deg, 1.0))
    wi = gw_ref[0, :D]
    wj = gw_ref[0, D:]
    b = gb_ref[0, 0]
    xv = x_ref[...]
    si = jnp.sum(xv * wi[None, :], axis=1) + b
    sj = jnp.sum(xv * wj[None, :], axis=1)
    pad = jnp.zeros((NPAD - N,), jnp.float32)
    sib_ref[...] = jnp.concatenate([si, pad])
    sj_ref[...] = jnp.concatenate([sj, pad])


_tc_node = pl.pallas_call(
    _tc_node_body,
    out_shape=[jax.ShapeDtypeStruct((NPAD,), jnp.float32)] * 3,
)


@functools.partial(
    pl.kernel,
    out_type=jax.ShapeDtypeStruct((E,), jnp.float32),
    mesh=_mesh,
    scratch_types=[
        pltpu.VMEM((NPAD,), jnp.float32),
        pltpu.VMEM((NPAD,), jnp.float32),
        pltpu.VMEM((NPAD,), jnp.float32),
        pltpu.VMEM((EPW,), jnp.int32),
        pltpu.VMEM((EPW,), jnp.int32),
        pltpu.VMEM((EPW,), jnp.float32),
    ],
    compiler_params=_sc_params,
)
def _sc_coef(dis_hbm, sib_hbm, sj_hbm, row_hbm, col_hbm, coef_hbm,
             dis_v, sib_v, sj_v, rbuf, cbuf, obuf):
    c = lax.axis_index("c")
    s = lax.axis_index("s")
    wid = s * NC + c
    ebase = wid * EPW
    pltpu.sync_copy(dis_hbm, dis_v)
    pltpu.sync_copy(sib_hbm, sib_v)
    pltpu.sync_copy(sj_hbm, sj_v)
    pltpu.sync_copy(row_hbm.at[pl.ds(ebase, EPW)], rbuf)
    pltpu.sync_copy(col_hbm.at[pl.ds(ebase, EPW)], cbuf)

    def grp(g, carry):
        o = g * 16
        r16 = rbuf[pl.ds(o, 16)]
        c16 = cbuf[pl.ds(o, 16)]
        z = plsc.load_gather(sib_v, [c16]) + plsc.load_gather(sj_v, [r16])
        alpha = 1.0 - 2.0 / (jnp.exp(2.0 * z) + 1.0)
        obuf[pl.ds(o, 16)] = (plsc.load_gather(dis_v, [r16])
                              * plsc.load_gather(dis_v, [c16]) * alpha)
        return carry

    lax.fori_loop(0, EPW // 16, grp, 0)
    pltpu.sync_copy(obuf, coef_hbm.at[pl.ds(ebase, EPW)])


@functools.partial(
    pl.kernel,
    out_type=jax.ShapeDtypeStruct((NC, NPAD, D), jnp.float32),
    mesh=_mesh,
    scratch_types=[
        [pltpu.VMEM((CH,), jnp.int32)] * 4,
        [pltpu.VMEM((CH,), jnp.int32)] * 4,
        [pltpu.VMEM((CH,), jnp.float32)] * 4,
        [pltpu.VMEM((CH, D // 2), jnp.int32)] * 2,
        [pltpu.VMEM((CH, D), jnp.float32)] * 2,
        pltpu.VMEM_SHARED((NPAD, D), jnp.float32),
        pltpu.SemaphoreType.DMA,
        pltpu.SemaphoreType.DMA,
        [pltpu.SemaphoreType.DMA] * 2,
    ],
    compiler_params=_sc_params_notile,
)
def _sc_edge(xi_hbm, row_hbm, col_hbm, coef_hbm, part_hbm,
             ri, ci, cf, bfb, rw, acc_sh, gsem, ssem, isem):
    c = lax.axis_index("c")
    s = lax.axis_index("s")
    wid = s * NC + c

    def zb(i, carry):
        for w in range(8):
            rw[0][i, pl.ds(w * 16, 16)] = jnp.zeros((16,), jnp.float32)
        return carry

    lax.fori_loop(0, CH, zb, 0)
    for t in range(RPT // CH):
        pltpu.sync_copy(rw[0], acc_sh.at[pl.ds(s * RPT + t * CH, CH)])
    plsc.subcore_barrier()

    ebase = wid * EPW

    def chunk_copies(kk, b):
        sl = pl.ds(ebase + kk * CH, CH)
        sem = isem[b % 2]
        return (
            (row_hbm.at[sl], ri[b], sem),
            (col_hbm.at[sl], ci[b], sem),
            (coef_hbm.at[sl], cf[b], sem),
        )

    def issue_idx(kk, b):
        for src, dst, sem in chunk_copies(kk, b):
            pltpu.async_copy(src, dst, sem)

    def wait_idx(kk, b):
        for src, dst, sem in chunk_copies(kk, b):
            pltpu.make_async_copy(src, dst, sem).wait()

    def scale_rows(b2, bq):
        def grp(g, carry):
            c16 = cf[bq][pl.ds(g * 16, 16)]
            for t in range(16):
                j = g * 16 + t
                cj = c16[t]
                for w in range(4):
                    wv = bfb[b2][j, pl.ds(w * 16, 16)]
                    v32 = plsc.bitcast(wv, jnp.bfloat16)
                    p0, p1 = plsc.unpack(v32, format=plsc.PackFormat.INTERLEAVED)
                    rw[b2][j, pl.ds(w * 32, 16)] = p0 * cj
                    rw[b2][j, pl.ds(w * 32 + 16, 16)] = p1 * cj
            return carry

        lax.fori_loop(0, CH // 16, grp, 0)

    issue_idx(0, 0)
    wait_idx(0, 0)
    pltpu.async_copy(xi_hbm.at[ri[0]], bfb[0], gsem)
    issue_idx(1, 1)

    def step(kk, b):
        b2 = b % 2
        pltpu.make_async_copy(xi_hbm.at[ri[b]], bfb[b2], gsem).wait()

        @pl.when(kk + 1 < NCHUNK)
        def _():
            b1 = (b + 1) % 4
            wait_idx(kk + 1, b1)
            pltpu.async_copy(xi_hbm.at[ri[b1]], bfb[1 - b2], gsem)

        scale_rows(b2, b)

        @pl.when(kk >= 1)
        def _():
            pltpu.make_async_copy(rw[1 - b2], acc_sh.at[ci[(b + 3) % 4]],
                                  ssem).wait()

        @pl.when(kk + 2 < NCHUNK)
        def _():
            issue_idx(kk + 2, (b + 2) % 4)

        pltpu.async_copy(rw[b2], acc_sh.at[ci[b]], ssem, add=True)

    def quad(q, carry):
        for b in range(4):
            kk = q * 4 + b

            @pl.when(kk < NCHUNK)
            def _():
                step(kk, b)

        return carry

    lax.fori_loop(0, (NCHUNK + 3) // 4, quad, 0)
    pltpu.make_async_copy(rw[0], acc_sh.at[ci[0]], ssem).wait()
    plsc.subcore_barrier()
    pltpu.sync_copy(acc_sh.at[pl.ds(s * RPT, RPT)],
                    part_hbm.at[c, pl.ds(s * RPT, RPT)])


def _tc_add_body(p_ref, o_ref):
    o_ref[...] = p_ref[0] + p_ref[1]


_tc_add = pl.pallas_call(
    _tc_add_body,
    out_shape=jax.ShapeDtypeStruct((N, D), jnp.float32),
    grid=(10,),
    in_specs=[pl.BlockSpec((NC, 1000, D), lambda i: (0, i, 0))],
    out_specs=pl.BlockSpec((1000, D), lambda i: (i, 0)),
)


def kernel(x, edge_index, gate_w, gate_b):
    x = x.astype(jnp.float32)
    ei = edge_index.astype(jnp.int32)
    row = ei[0]
    col = ei[1]
    col_pad = jnp.concatenate(
        [col, jnp.full((NW * HRPT * 128 - E,), NPAD - 1, jnp.int32)]
    ).reshape(NW * HRPT, 128)
    deg2 = _sc_hist(col_pad)
    dis, sib, sj = _tc_node(deg2, x, gate_w, gate_b.reshape(1, 1))
    coef = _sc_coef(dis, sib, sj, row, col)
    xp = x.astype(jnp.bfloat16).reshape(N, 4, 2, 16).swapaxes(2, 3)
    xi = jax.lax.bitcast_convert_type(xp.reshape(N, D // 2, 2), jnp.int32)
    parts = _sc_edge(xi, row, col, coef)
    return _tc_add(parts)


# speedup vs baseline: 1.6019x; 1.6019x over previous
import functools

import jax
import jax.numpy as jnp
from jax import lax
from jax.experimental import pallas as pl
from jax.experimental.pallas import tpu as pltpu
from jax.experimental.pallas import tpu_sc as plsc

N = 10000
E = 320000
D = 128
NPAD = 10240
NC, NS = 2, 16
NW = NC * NS
EPW = E // NW
CH = 80
NCHUNK = EPW // CH
RPT = NPAD // NS
HRPT = 80

_mesh = plsc.VectorSubcoreMesh(core_axis_name="c", subcore_axis_name="s")
_sc_params = pltpu.CompilerParams(needs_layout_passes=False)


@functools.partial(
    pl.kernel,
    out_type=[jax.ShapeDtypeStruct((NPAD,), jnp.float32)] * 2,
    mesh=_mesh,
    scratch_types=[
        pltpu.VMEM((HRPT, 128), jnp.int32),
        pltpu.VMEM((128,), jnp.float32),
        pltpu.VMEM((RPT,), jnp.float32),
        pltpu.VMEM_SHARED((NPAD,), jnp.float32),
    ],
    compiler_params=_sc_params,
)
def _sc_hist(col_hbm, out0_hbm, out1_hbm, colblk, ones_v, zb_v, hist_sh):
    c = lax.axis_index("c")
    s = lax.axis_index("s")
    wid = s * NC + c
    for g in range(8):
        ones_v[pl.ds(g * 16, 16)] = jnp.full((16,), 1.0, jnp.float32)
    for g in range(RPT // 16):
        zb_v[pl.ds(g * 16, 16)] = jnp.zeros((16,), jnp.float32)
    pltpu.sync_copy(col_hbm.at[pl.ds(wid * HRPT, HRPT)], colblk)
    pltpu.sync_copy(zb_v, hist_sh.at[pl.ds(s * RPT, RPT)])
    plsc.subcore_barrier()

    def body(j, carry):
        pltpu.sync_copy(ones_v, hist_sh.at[colblk.at[j]], add=True)
        return carry

    lax.fori_loop(0, HRPT, body, 0)
    plsc.subcore_barrier()

    @pl.when(c == 0)
    def _():
        pltpu.sync_copy(hist_sh.at[pl.ds(s * RPT, RPT)],
                        out0_hbm.at[pl.ds(s * RPT, RPT)])

    @pl.when(c == 1)
    def _():
        pltpu.sync_copy(hist_sh.at[pl.ds(s * RPT, RPT)],
                        out1_hbm.at[pl.ds(s * RPT, RPT)])


def _tc_sisj_body(x_ref, gw_ref, gb_ref, sib_ref, sj_ref):
    wi = gw_ref[0, :D]
    wj = gw_ref[0, D:]
    b = gb_ref[0, 0]
    xv = x_ref[...]
    si = jnp.sum(xv * wi[None, :], axis=1) + b
    sj = jnp.sum(xv * wj[None, :], axis=1)
    pad = jnp.zeros((NPAD - N,), jnp.float32)
    sib_ref[...] = jnp.concatenate([si, pad])
    sj_ref[...] = jnp.concatenate([sj, pad])


_tc_sisj = pl.pallas_call(
    _tc_sisj_body,
    out_shape=[jax.ShapeDtypeStruct((NPAD,), jnp.float32)] * 2,
)


@functools.partial(
    pl.kernel,
    out_type=jax.ShapeDtypeStruct((E,), jnp.float32),
    mesh=_mesh,
    scratch_types=[
        pltpu.VMEM((NPAD,), jnp.float32),
        pltpu.VMEM((NPAD,), jnp.float32),
        pltpu.VMEM((NPAD,), jnp.float32),
        pltpu.VMEM((NPAD,), jnp.float32),
        pltpu.VMEM((EPW,), jnp.int32),
        pltpu.VMEM((EPW,), jnp.int32),
        pltpu.VMEM((EPW,), jnp.float32),
    ],
    compiler_params=_sc_params,
)
def _sc_coef(p0_hbm, p1_hbm, sib_hbm, sj_hbm, row_hbm, col_hbm, coef_hbm,
             dis_v, tmp_v, sib_v, sj_v, rbuf, cbuf, obuf):
    c = lax.axis_index("c")
    s = lax.axis_index("s")
    wid = s * NC + c
    ebase = wid * EPW
    pltpu.sync_copy(p0_hbm, dis_v)
    pltpu.sync_copy(p1_hbm, tmp_v)
    pltpu.sync_copy(sib_hbm, sib_v)
    pltpu.sync_copy(sj_hbm, sj_v)
    pltpu.sync_copy(row_hbm.at[pl.ds(ebase, EPW)], rbuf)
    pltpu.sync_copy(col_hbm.at[pl.ds(ebase, EPW)], cbuf)

    def rsq(g, carry):
        o = g * 16
        d = jnp.maximum(dis_v[pl.ds(o, 16)] + tmp_v[pl.ds(o, 16)], 1.0)
        i0 = jnp.full((16,), 0x5F3759DF, jnp.int32) - lax.shift_right_logical(
            plsc.bitcast(d, jnp.int32), 1)
        y = plsc.bitcast(i0, jnp.float32)
        for _ in range(3):
            y = y * (1.5 - 0.5 * d * y * y)
        dis_v[pl.ds(o, 16)] = y
        return carry

    lax.fori_loop(0, NPAD // 16, rsq, 0)

    def grp(g, carry):
        o = g * 16
        r16 = rbuf[pl.ds(o, 16)]
        c16 = cbuf[pl.ds(o, 16)]
        z = plsc.load_gather(sib_v, [c16]) + plsc.load_gather(sj_v, [r16])
        alpha = 1.0 - 2.0 / (jnp.exp(2.0 * z) + 1.0)
        obuf[pl.ds(o, 16)] = (plsc.load_gather(dis_v, [r16])
                              * plsc.load_gather(dis_v, [c16]) * alpha)
        return carry

    lax.fori_loop(0, EPW // 16, grp, 0)
    pltpu.sync_copy(obuf, coef_hbm.at[pl.ds(ebase, EPW)])


@functools.partial(
    pl.kernel,
    out_type=jax.ShapeDtypeStruct((NC, NPAD, D), jnp.float32),
    mesh=_mesh,
    scratch_types=[
        [pltpu.VMEM((CH,), jnp.int32)] * 4,
        [pltpu.VMEM((CH,), jnp.int32)] * 4,
        [pltpu.VMEM((CH,), jnp.float32)] * 4,
        [pltpu.VMEM((CH, D), jnp.float32)] * 4,
        pltpu.VMEM_SHARED((NPAD, D), jnp.float32),
        pltpu.SemaphoreType.DMA,
        [pltpu.SemaphoreType.DMA] * 2,
        [pltpu.SemaphoreType.DMA] * 2,
    ],
    compiler_params=_sc_params,
)
def _sc_edge(x_hbm, row_hbm, col_hbm, coef_hbm, part_hbm,
             ri, ci, cf, rw, acc_sh, gsem, ssem, isem):
    c = lax.axis_index("c")
    s = lax.axis_index("s")
    wid = s * NC + c

    def zb(i, carry):
        for w in range(8):
            rw[0][i, pl.ds(w * 16, 16)] = jnp.zeros((16,), jnp.float32)
        return carry

    lax.fori_loop(0, CH, zb, 0)
    for t in range(RPT // CH):
        pltpu.sync_copy(rw[0], acc_sh.at[pl.ds(s * RPT + t * CH, CH)])
    plsc.subcore_barrier()

    ebase = wid * EPW

    def chunk_copies(kk, b):
        sl = pl.ds(ebase + kk * CH, CH)
        sem = isem[b % 2]
        return (
            (row_hbm.at[sl], ri[b], sem),
            (col_hbm.at[sl], ci[b], sem),
            (coef_hbm.at[sl], cf[b], sem),
        )

    def issue_idx(kk, b):
        for src, dst, sem in chunk_copies(kk, b):
            pltpu.async_copy(src, dst, sem)

    def wait_idx(kk, b):
        for src, dst, sem in chunk_copies(kk, b):
            pltpu.make_async_copy(src, dst, sem).wait()

    def scale_rows(b):
        def grp(g, carry):
            c16 = cf[b][pl.ds(g * 16, 16)]
            for t in range(16):
                j = g * 16 + t
                cj = c16[t]
                for w in range(8):
                    rw[b][j, pl.ds(w * 16, 16)] = rw[b][j, pl.ds(w * 16, 16)] * cj
            return carry

        lax.fori_loop(0, CH // 16, grp, 0)

    issue_idx(0, 0)
    wait_idx(0, 0)
    pltpu.async_copy(x_hbm.at[ri[0]], rw[0], gsem)
    issue_idx(1, 1)

    def step(kk, b):
        pltpu.make_async_copy(x_hbm.at[ri[b]], rw[b], gsem).wait()

        @pl.when(kk + 1 < NCHUNK)
        def _():
            b1 = (b + 1) % 4
            wait_idx(kk + 1, b1)
            pltpu.async_copy(x_hbm.at[ri[b1]], rw[b1], gsem)

        scale_rows(b)

        @pl.when(kk >= 2)
        def _():
            b2 = (b + 2) % 4
            pltpu.make_async_copy(rw[b2], acc_sh.at[ci[b2]], ssem[b % 2]).wait()

        @pl.when(kk + 2 < NCHUNK)
        def _():
            issue_idx(kk + 2, (b + 2) % 4)

        pltpu.async_copy(rw[b], acc_sh.at[ci[b]], ssem[b % 2], add=True)

    def quad(q, carry):
        for b in range(4):
            kk = q * 4 + b

            @pl.when(kk < NCHUNK)
            def _():
                step(kk, b)

        return carry

    lax.fori_loop(0, (NCHUNK + 3) // 4, quad, 0)
    pltpu.make_async_copy(rw[3], acc_sh.at[ci[3]], ssem[1]).wait()
    pltpu.make_async_copy(rw[0], acc_sh.at[ci[0]], ssem[0]).wait()
    plsc.subcore_barrier()
    pltpu.sync_copy(acc_sh.at[pl.ds(s * RPT, RPT)],
                    part_hbm.at[c, pl.ds(s * RPT, RPT)])


def _tc_add_body(p_ref, o_ref):
    o_ref[...] = p_ref[0] + p_ref[1]


_tc_add = pl.pallas_call(
    _tc_add_body,
    out_shape=jax.ShapeDtypeStruct((N, D), jnp.float32),
    grid=(10,),
    in_specs=[pl.BlockSpec((NC, 1000, D), lambda i: (0, i, 0))],
    out_specs=pl.BlockSpec((1000, D), lambda i: (i, 0)),
)


def kernel(x, edge_index, gate_w, gate_b):
    x = x.astype(jnp.float32)
    ei = edge_index.astype(jnp.int32)
    row = ei[0]
    col = ei[1]
    col_pad = jnp.concatenate(
        [col, jnp.full((NW * HRPT * 128 - E,), NPAD - 1, jnp.int32)]
    ).reshape(NW * HRPT, 128)
    p0, p1 = _sc_hist(col_pad)
    sib, sj = _tc_sisj(x, gate_w, gate_b.reshape(1, 1))
    coef = _sc_coef(p0, p1, sib, sj, row, col)
    parts = _sc_edge(x, row, col, coef)
    return _tc_add(parts)


# speedup vs baseline: 1.7187x; 1.0729x over previous
import functools

import jax
import jax.numpy as jnp
from jax import lax
from jax.experimental import pallas as pl
from jax.experimental.pallas import tpu as pltpu
from jax.experimental.pallas import tpu_sc as plsc

N = 10000
E = 320000
D = 128
NPAD = 10240
NC, NS = 2, 16
NW = NC * NS
EPW = E // NW
CH = 80
NCHUNK = EPW // CH
RPT = NPAD // NS
HRPT = 80

_mesh = plsc.VectorSubcoreMesh(core_axis_name="c", subcore_axis_name="s")
_sc_params = pltpu.CompilerParams(needs_layout_passes=False)


@functools.partial(
    pl.kernel,
    out_type=[jax.ShapeDtypeStruct((NPAD,), jnp.float32)] * 2,
    mesh=_mesh,
    scratch_types=[
        pltpu.VMEM((HRPT, 128), jnp.int32),
        pltpu.VMEM((128,), jnp.float32),
        pltpu.VMEM((RPT,), jnp.float32),
        pltpu.VMEM_SHARED((NPAD,), jnp.float32),
    ],
    compiler_params=_sc_params,
)
def _sc_hist(col_hbm, out0_hbm, out1_hbm, colblk, ones_v, zb_v, hist_sh):
    c = lax.axis_index("c")
    s = lax.axis_index("s")
    wid = s * NC + c
    for g in range(8):
        ones_v[pl.ds(g * 16, 16)] = jnp.full((16,), 1.0, jnp.float32)
    for g in range(RPT // 16):
        zb_v[pl.ds(g * 16, 16)] = jnp.zeros((16,), jnp.float32)
    pltpu.sync_copy(col_hbm.at[pl.ds(wid * HRPT, HRPT)], colblk)
    pltpu.sync_copy(zb_v, hist_sh.at[pl.ds(s * RPT, RPT)])
    plsc.subcore_barrier()

    def body(j, carry):
        pltpu.sync_copy(ones_v, hist_sh.at[colblk.at[j]], add=True)
        return carry

    lax.fori_loop(0, HRPT, body, 0)
    plsc.subcore_barrier()

    @pl.when(c == 0)
    def _():
        pltpu.sync_copy(hist_sh.at[pl.ds(s * RPT, RPT)],
                        out0_hbm.at[pl.ds(s * RPT, RPT)])

    @pl.when(c == 1)
    def _():
        pltpu.sync_copy(hist_sh.at[pl.ds(s * RPT, RPT)],
                        out1_hbm.at[pl.ds(s * RPT, RPT)])


def _tc_sisj_body(x_ref, gw_ref, gb_ref, sib_ref, sj_ref):
    wi = gw_ref[0, :D]
    wj = gw_ref[0, D:]
    b = gb_ref[0, 0]
    xv = x_ref[...]
    si = jnp.sum(xv * wi[None, :], axis=1) + b
    sj = jnp.sum(xv * wj[None, :], axis=1)
    pad = jnp.zeros((NPAD - N,), jnp.float32)
    sib_ref[...] = jnp.concatenate([si, pad])
    sj_ref[...] = jnp.concatenate([sj, pad])


_tc_sisj = pl.pallas_call(
    _tc_sisj_body,
    out_shape=[jax.ShapeDtypeStruct((NPAD,), jnp.float32)] * 2,
)


@functools.partial(
    pl.kernel,
    out_type=jax.ShapeDtypeStruct((E,), jnp.float32),
    mesh=_mesh,
    scratch_types=[
        pltpu.VMEM((NPAD,), jnp.float32),
        pltpu.VMEM((RPT,), jnp.float32),
        pltpu.VMEM((RPT,), jnp.float32),
        pltpu.VMEM((NPAD,), jnp.float32),
        pltpu.VMEM((NPAD,), jnp.float32),
        pltpu.VMEM((EPW,), jnp.int32),
        pltpu.VMEM((EPW,), jnp.int32),
        pltpu.VMEM((EPW,), jnp.float32),
        pltpu.VMEM_SHARED((NPAD,), jnp.float32),
        pltpu.SemaphoreType.DMA,
    ],
    compiler_params=_sc_params,
)
def _sc_coef(p0_hbm, p1_hbm, sib_hbm, sj_hbm, row_hbm, col_hbm, coef_hbm,
             dis_v, a_v, b_v, sib_v, sj_v, rbuf, cbuf, obuf, dis_sh, sem):
    c = lax.axis_index("c")
    s = lax.axis_index("s")
    wid = s * NC + c
    ebase = wid * EPW
    big_copies = (
        (sib_hbm, sib_v),
        (sj_hbm, sj_v),
        (row_hbm.at[pl.ds(ebase, EPW)], rbuf),
        (col_hbm.at[pl.ds(ebase, EPW)], cbuf),
    )
    for src, dst in big_copies:
        pltpu.async_copy(src, dst, sem)

    nbase = s * RPT
    pltpu.sync_copy(p0_hbm.at[pl.ds(nbase, RPT)], a_v)
    pltpu.sync_copy(p1_hbm.at[pl.ds(nbase, RPT)], b_v)

    def rsq(g, carry):
        o = g * 16
        d = jnp.maximum(a_v[pl.ds(o, 16)] + b_v[pl.ds(o, 16)], 1.0)
        i0 = jnp.full((16,), 0x5F3759DF, jnp.int32) - lax.shift_right_logical(
            plsc.bitcast(d, jnp.int32), 1)
        y = plsc.bitcast(i0, jnp.float32)
        for _ in range(3):
            y = y * (1.5 - 0.5 * d * y * y)
        a_v[pl.ds(o, 16)] = y
        return carry

    lax.fori_loop(0, RPT // 16, rsq, 0)
    pltpu.sync_copy(a_v, dis_sh.at[pl.ds(nbase, RPT)])
    plsc.subcore_barrier()
    pltpu.sync_copy(dis_sh, dis_v)
    for src, dst in big_copies:
        pltpu.make_async_copy(src, dst, sem).wait()

    def grp(g, carry):
        for u in range(2):
            o = g * 32 + u * 16
            r16 = rbuf[pl.ds(o, 16)]
            c16 = cbuf[pl.ds(o, 16)]
            z = plsc.load_gather(sib_v, [c16]) + plsc.load_gather(sj_v, [r16])
            alpha = 1.0 - 2.0 / (jnp.exp(2.0 * z) + 1.0)
            obuf[pl.ds(o, 16)] = (plsc.load_gather(dis_v, [r16])
                                  * plsc.load_gather(dis_v, [c16]) * alpha)
        return carry

    lax.fori_loop(0, EPW // 32, grp, 0)
    pltpu.sync_copy(obuf, coef_hbm.at[pl.ds(ebase, EPW)])


@functools.partial(
    pl.kernel,
    out_type=jax.ShapeDtypeStruct((NC, NPAD, D), jnp.float32),
    mesh=_mesh,
    scratch_types=[
        [pltpu.VMEM((CH,), jnp.int32)] * 4,
        [pltpu.VMEM((CH,), jnp.int32)] * 4,
        [pltpu.VMEM((CH,), jnp.float32)] * 4,
        [pltpu.VMEM((CH, D), jnp.float32)] * 4,
        pltpu.VMEM_SHARED((NPAD, D), jnp.float32),
        pltpu.SemaphoreType.DMA,
        [pltpu.SemaphoreType.DMA] * 2,
        [pltpu.SemaphoreType.DMA] * 2,
    ],
    compiler_params=_sc_params,
)
def _sc_edge(x_hbm, row_hbm, col_hbm, coef_hbm, part_hbm,
             ri, ci, cf, rw, acc_sh, gsem, ssem, isem):
    c = lax.axis_index("c")
    s = lax.axis_index("s")
    wid = s * NC + c

    def zb(i, carry):
        for w in range(8):
            rw[0][i, pl.ds(w * 16, 16)] = jnp.zeros((16,), jnp.float32)
        return carry

    lax.fori_loop(0, CH, zb, 0)
    for t in range(RPT // CH):
        pltpu.sync_copy(rw[0], acc_sh.at[pl.ds(s * RPT + t * CH, CH)])
    plsc.subcore_barrier()

    ebase = wid * EPW

    def chunk_copies(kk, b):
        sl = pl.ds(ebase + kk * CH, CH)
        sem = isem[b % 2]
        return (
            (row_hbm.at[sl], ri[b], sem),
            (col_hbm.at[sl], ci[b], sem),
            (coef_hbm.at[sl], cf[b], sem),
        )

    def issue_idx(kk, b):
        for src, dst, sem in chunk_copies(kk, b):
            pltpu.async_copy(src, dst, sem)

    def wait_idx(kk, b):
        for src, dst, sem in chunk_copies(kk, b):
            pltpu.make_async_copy(src, dst, sem).wait()

    def scale_rows(b):
        def grp(g, carry):
            c16 = cf[b][pl.ds(g * 16, 16)]
            for t in range(16):
                j = g * 16 + t
                cj = c16[t]
                for w in range(8):
                    rw[b][j, pl.ds(w * 16, 16)] = rw[b][j, pl.ds(w * 16, 16)] * cj
            return carry

        lax.fori_loop(0, CH // 16, grp, 0)

    issue_idx(0, 0)
    wait_idx(0, 0)
    pltpu.async_copy(x_hbm.at[ri[0]], rw[0], gsem)
    issue_idx(1, 1)

    def step(kk, b):
        pltpu.make_async_copy(x_hbm.at[ri[b]], rw[b], gsem).wait()

        @pl.when(kk + 1 < NCHUNK)
        def _():
            b1 = (b + 1) % 4
            wait_idx(kk + 1, b1)
            pltpu.async_copy(x_hbm.at[ri[b1]], rw[b1], gsem)

        scale_rows(b)

        @pl.when(kk >= 2)
        def _():
            b2 = (b + 2) % 4
            pltpu.make_async_copy(rw[b2], acc_sh.at[ci[b2]], ssem[b % 2]).wait()

        @pl.when(kk + 2 < NCHUNK)
        def _():
            issue_idx(kk + 2, (b + 2) % 4)

        pltpu.async_copy(rw[b], acc_sh.at[ci[b]], ssem[b % 2], add=True)

    def quad(q, carry):
        for b in range(4):
            kk = q * 4 + b

            @pl.when(kk < NCHUNK)
            def _():
                step(kk, b)

        return carry

    lax.fori_loop(0, (NCHUNK + 3) // 4, quad, 0)
    pltpu.make_async_copy(rw[3], acc_sh.at[ci[3]], ssem[1]).wait()
    pltpu.make_async_copy(rw[0], acc_sh.at[ci[0]], ssem[0]).wait()
    plsc.subcore_barrier()
    pltpu.sync_copy(acc_sh.at[pl.ds(s * RPT, RPT)],
                    part_hbm.at[c, pl.ds(s * RPT, RPT)])


def _tc_add_body(p_ref, o_ref):
    o_ref[...] = p_ref[0] + p_ref[1]


_tc_add = pl.pallas_call(
    _tc_add_body,
    out_shape=jax.ShapeDtypeStruct((N, D), jnp.float32),
    grid=(10,),
    in_specs=[pl.BlockSpec((NC, 1000, D), lambda i: (0, i, 0))],
    out_specs=pl.BlockSpec((1000, D), lambda i: (i, 0)),
)


def kernel(x, edge_index, gate_w, gate_b):
    x = x.astype(jnp.float32)
    ei = edge_index.astype(jnp.int32)
    row = ei[0]
    col = ei[1]
    col_pad = jnp.concatenate(
        [col, jnp.full((NW * HRPT * 128 - E,), NPAD - 1, jnp.int32)]
    ).reshape(NW * HRPT, 128)
    p0, p1 = _sc_hist(col_pad)
    sib, sj = _tc_sisj(x, gate_w, gate_b.reshape(1, 1))
    coef = _sc_coef(p0, p1, sib, sj, row, col)
    parts = _sc_edge(x, row, col, coef)
    return _tc_add(parts)
